# baseline (speedup 1.0000x reference)
import jax
import jax.numpy as jnp
from jax.experimental import pallas as pl

_INDEX = 50
_ROWS = 200
_FEAT = 64
_COLS = _ROWS * _FEAT
_COL0 = _INDEX * _FEAT
_BB = 64


def _select_scatter_block(x_ref, src_ref, o_ref):
    o_ref[...] = x_ref[...]
    o_ref[:, _COL0:_COL0 + _FEAT] = src_ref[...]


def kernel(x, src):
    b = x.shape[0]
    x2 = x.reshape(b, _COLS)
    out = pl.pallas_call(
        _select_scatter_block,
        grid=(b // _BB,),
        in_specs=[
            pl.BlockSpec((_BB, _COLS), lambda i: (i, 0)),
            pl.BlockSpec((_BB, _FEAT), lambda i: (i, 0)),
        ],
        out_specs=pl.BlockSpec((_BB, _COLS), lambda i: (i, 0)),
        out_shape=jax.ShapeDtypeStruct((b, _COLS), x.dtype),
    )(x2, src)
    return out.reshape(x.shape)
